# Initial kernel scaffold; baseline (speedup 1.0000x reference)
import jax
import jax.numpy as jnp
from jax.experimental import pallas as pl


def kernel(x, y):
    raise NotImplementedError("write your pallas kernel here")


# speedup vs baseline: 1.8996x; 1.8996x over previous
import jax
import jax.numpy as jnp
from jax.experimental import pallas as pl
from jax.experimental.pallas import tpu as pltpu

_PATCH = 7
_STRIDE = 2
_ALPHA = 0.05
_N = 2025
_NPAD = 2048
_D = 147
_DPAD = 256
_BIG = 1.0e30


def _extract_patches(x):
    patches = jax.lax.conv_general_dilated_patches(
        x, filter_shape=(_PATCH, _PATCH), window_strides=(_STRIDE, _STRIDE),
        padding='VALID')
    b, d, hh, ww = patches.shape
    return patches.reshape(b, d, hh * ww).transpose(0, 2, 1)


def _loss_kernel(inp_ref, tgt_ref, out_ref):
    b = pl.program_id(0)
    inp = inp_ref[0]
    tgt = tgt_ref[0]

    xn = jnp.sum(inp * inp, axis=1)
    yn = jnp.sum(tgt * tgt, axis=1)
    cross = jax.lax.dot_general(
        tgt, inp, (((1,), (1,)), ((), ())),
        preferred_element_type=jnp.float32)

    col_ids = jax.lax.broadcasted_iota(jnp.int32, (_NPAD, _NPAD), 1)
    row_ids = jax.lax.broadcasted_iota(jnp.int32, (_NPAD, _NPAD), 0)

    dist = (yn[:, None] + xn[None, :] - 2.0 * cross) * (1.0 / _D)
    dist = jnp.where(col_ids < _N, dist, _BIG)
    rowmin = jnp.min(dist, axis=1)
    norm = dist / (rowmin + _ALPHA)[:, None]
    norm = jnp.where(row_ids < _N, norm, _BIG)
    colmin = jnp.min(norm, axis=0, keepdims=True)
    colmask = jax.lax.broadcasted_iota(jnp.int32, (1, _NPAD), 1) < _N
    loss_b = jnp.sum(jnp.where(colmask, colmin, 0.0),
                     axis=1, keepdims=True) * (1.0 / _N)

    @pl.when(b == 0)
    def _init():
        out_ref[...] = jnp.zeros_like(out_ref)
    out_ref[...] += loss_b * 0.5


def kernel(x, y):
    xp = _extract_patches(x)
    yp = _extract_patches(y)
    bsz = xp.shape[0]
    xp = jnp.pad(xp, ((0, 0), (0, _NPAD - _N), (0, _DPAD - _D)))
    yp = jnp.pad(yp, ((0, 0), (0, _NPAD - _N), (0, _DPAD - _D)))

    out = pl.pallas_call(
        _loss_kernel,
        grid=(bsz,),
        in_specs=[
            pl.BlockSpec((1, _NPAD, _DPAD), lambda b: (b, 0, 0)),
            pl.BlockSpec((1, _NPAD, _DPAD), lambda b: (b, 0, 0)),
        ],
        out_specs=pl.BlockSpec((1, 1), lambda b: (0, 0)),
        out_shape=jax.ShapeDtypeStruct((1, 1), jnp.float32),
    )(xp, yp)
    return out[0, 0]
